# baseline (speedup 1.0000x reference)
import functools

import jax
import jax.numpy as jnp
from jax import lax
from jax.experimental import pallas as pl
from jax.experimental.pallas import tpu as pltpu
from jax.experimental.pallas import tpu_sc as plsc

N_ROWS = 8192
D = 1024

SC_ROWS = 2048
NUM_CORES = 2
NUM_SUBCORES = 16
NW = NUM_CORES * NUM_SUBCORES
PER_W = SC_ROWS * D // NW
CHUNK = 16384
NCHUNK = PER_W // CHUNK
GROUPS = CHUNK // 16

TC_BLOCK = 1024
TC_BLOCK0 = SC_ROWS // TC_BLOCK
TC_STEPS = (N_ROWS - SC_ROWS) // TC_BLOCK


def _sc_partial_kernel(g1, g1h, g2, g2h, out, b1, b2, b3, b4):
    wid = lax.axis_index("s") * NUM_CORES + lax.axis_index("c")
    base = wid * PER_W

    def chunk_body(c, accs):
        off = base + c * CHUNK
        pltpu.sync_copy(g1.at[pl.ds(off, CHUNK)], b1)
        pltpu.sync_copy(g1h.at[pl.ds(off, CHUNK)], b2)
        pltpu.sync_copy(g2.at[pl.ds(off, CHUNK)], b3)
        pltpu.sync_copy(g2h.at[pl.ds(off, CHUNK)], b4)

        def grp(i, accs2):
            x1, x2 = accs2
            d1 = b1[pl.ds(i * 16, 16)] - b2[pl.ds(i * 16, 16)]
            d2 = b3[pl.ds(i * 16, 16)] - b4[pl.ds(i * 16, 16)]
            return (x1 + d1 * d1, x2 + d2 * d2)

        return lax.fori_loop(0, GROUPS, grp, accs)

    zero = jnp.zeros((16,), jnp.float32)
    a1, a2 = lax.fori_loop(0, NCHUNK, chunk_body, (zero, zero))
    b1[pl.ds(0, 16)] = a1 + a2
    pltpu.sync_copy(b1.at[pl.ds(0, 16)], out.at[pl.ds(wid * 16, 16)])


def _tc_sum_kernel(g1_ref, g2_ref, g1h_ref, g2h_ref, out_ref):
    i = pl.program_id(0)
    d1 = g1_ref[...] - g1h_ref[...]
    d2 = g2_ref[...] - g2h_ref[...]
    partial = jnp.sum(d1 * d1) + jnp.sum(d2 * d2)

    @pl.when(i == 0)
    def _init():
        out_ref[0] = 0.0

    out_ref[0] += partial


def kernel(x, target, g1, g2, g1_hat, g2_hat):
    g1f = g1.reshape(-1)
    g2f = g2.reshape(-1)
    g1hf = g1_hat.reshape(-1)
    g2hf = g2_hat.reshape(-1)

    sc_fn = functools.partial(
        pl.kernel,
        out_type=jax.ShapeDtypeStruct((NW * 16,), jnp.float32),
        mesh=plsc.VectorSubcoreMesh(core_axis_name="c", subcore_axis_name="s"),
        scratch_types=[
            pltpu.VMEM((CHUNK,), jnp.float32),
            pltpu.VMEM((CHUNK,), jnp.float32),
            pltpu.VMEM((CHUNK,), jnp.float32),
            pltpu.VMEM((CHUNK,), jnp.float32),
        ],
    )(_sc_partial_kernel)
    sc_partials = sc_fn(g1f, g1hf, g2f, g2hf)

    spec = pl.BlockSpec((TC_BLOCK, D), lambda i: (i + TC_BLOCK0, 0))
    tc_total = pl.pallas_call(
        _tc_sum_kernel,
        grid=(TC_STEPS,),
        in_specs=[spec, spec, spec, spec],
        out_specs=pl.BlockSpec(memory_space=pltpu.SMEM),
        out_shape=jax.ShapeDtypeStruct((1,), jnp.float32),
    )(g1, g2, g1_hat, g2_hat)

    total = tc_total[0] + jnp.sum(sc_partials)
    return total * jnp.float32(1.0 / (N_ROWS * D))


# speedup vs baseline: 4.4025x; 4.4025x over previous
import jax
import jax.numpy as jnp
from jax.experimental import pallas as pl
from jax.experimental.pallas import tpu as pltpu


def _make_mse_kernel(num_blocks, inv_count):
    def _mse_mean_kernel(g1_ref, g2_ref, g1h_ref, g2h_ref, out_ref):
        i = pl.program_id(0)
        d1 = g1_ref[...] - g1h_ref[...]
        d2 = g2_ref[...] - g2h_ref[...]
        partial = jnp.sum(d1 * d1) + jnp.sum(d2 * d2)

        @pl.when(i == 0)
        def _init():
            out_ref[0] = 0.0

        out_ref[0] += partial

        @pl.when(i == num_blocks - 1)
        def _finish():
            out_ref[0] = out_ref[0] * inv_count

    return _mse_mean_kernel


def kernel(x, target, g1, g2, g1_hat, g2_hat):
    N, D = g1.shape
    BLOCK = 512
    grid = (N // BLOCK,)
    spec = pl.BlockSpec((BLOCK, D), lambda i: (i, 0))
    total = pl.pallas_call(
        _make_mse_kernel(N // BLOCK, 1.0 / (N * D)),
        grid=grid,
        in_specs=[spec, spec, spec, spec],
        out_specs=pl.BlockSpec(memory_space=pltpu.SMEM),
        out_shape=jax.ShapeDtypeStruct((1,), jnp.float32),
    )(g1, g2, g1_hat, g2_hat)
    return total[0]
